# baseline (speedup 1.0000x reference)
import functools

import jax
import jax.numpy as jnp
from jax import lax
from jax.experimental import pallas as pl
from jax.experimental.pallas import tpu as pltpu
from jax.experimental.pallas import tpu_sc as plsc

_LANES = 16
_CHUNK = 128


@functools.lru_cache(maxsize=None)
def _make_sc_logits(B, V, D, NEG):
    info = plsc.get_sparse_core_info()
    NC, NS = info.num_cores, info.num_subcores
    NW = NC * NS
    per_w = B // NW
    n_chunks = per_w // _CHUNK
    OUTW = NEG + 1
    H = D // _LANES
    mesh = plsc.VectorSubcoreMesh(core_axis_name="c", subcore_axis_name="s")

    def body(inputs_hbm, labels_hbm, sampled_hbm, bidx_hbm,
             syn0_hbm, syn1_hbm, biases_hbm, out_hbm,
             in_idx, lab_idx, samp_idx, bidx_v,
             a_rows, t_rows, s_rows, bias_v, out_v, sem):
        wid = lax.axis_index("s") * NC + lax.axis_index("c")
        lane = lax.iota(jnp.int32, _LANES)
        sign = jnp.where(lane == 0, -1.0, 1.0).astype(jnp.float32)

        for c in range(n_chunks):
            base = wid * per_w + c * _CHUNK
            pltpu.sync_copy(inputs_hbm.at[pl.ds(base, _CHUNK)], in_idx)
            pltpu.sync_copy(labels_hbm.at[pl.ds(base, _CHUNK)], lab_idx)
            pltpu.sync_copy(
                sampled_hbm.at[pl.ds(base * NEG, NEG * _CHUNK)], samp_idx)
            pltpu.sync_copy(
                bidx_hbm.at[pl.ds(base * OUTW, OUTW * _CHUNK)], bidx_v)
            cps = [
                pltpu.async_copy(syn0_hbm.at[in_idx], a_rows, sem),
                pltpu.async_copy(syn1_hbm.at[lab_idx], t_rows, sem),
            ]
            for j in range(NEG):
                cps.append(pltpu.async_copy(
                    syn1_hbm.at[samp_idx.at[pl.ds(j * _CHUNK, _CHUNK)]],
                    s_rows.at[pl.ds(j * _CHUNK, _CHUNK)], sem))
            for j in range(OUTW):
                cps.append(pltpu.async_copy(
                    biases_hbm.at[bidx_v.at[pl.ds(j * _CHUNK, _CHUNK)]],
                    bias_v.at[pl.ds(j * _CHUNK, _CHUNK)], sem))
            for cp in cps:
                cp.wait()

            def elem(e, carry):
                a = [a_rows[e, pl.ds(h * _LANES, _LANES)] for h in range(H)]
                p = a[0] * t_rows[e, pl.ds(0, _LANES)]
                for h in range(1, H):
                    p = p + a[h] * t_rows[e, pl.ds(h * _LANES, _LANES)]
                res = jnp.where(lane == 0, jnp.sum(p), 0.0)
                for t in range(NEG):
                    r = e * NEG + t
                    p = a[0] * s_rows[r, pl.ds(0, _LANES)]
                    for h in range(1, H):
                        p = p + a[h] * s_rows[r, pl.ds(h * _LANES, _LANES)]
                    res = jnp.where(lane == t + 1, jnp.sum(p), res)
                bvec = bias_v[pl.ds(e * OUTW, _LANES)]
                plsc.store_scatter(out_v, [e * OUTW + lane],
                                   sign * (res + bvec), mask=lane < OUTW)
                return carry

            lax.fori_loop(0, _CHUNK, elem, 0)
            pltpu.sync_copy(out_v,
                            out_hbm.at[pl.ds(base * OUTW, _CHUNK * OUTW)])

    return pl.kernel(
        body,
        mesh=mesh,
        out_type=jax.ShapeDtypeStruct((B * OUTW,), jnp.float32),
        compiler_params=pltpu.CompilerParams(
            needs_layout_passes=False, use_tc_tiling_on_sc=False),
        scratch_types=[
            pltpu.VMEM((_CHUNK,), jnp.int32),
            pltpu.VMEM((_CHUNK,), jnp.int32),
            pltpu.VMEM((NEG * _CHUNK,), jnp.int32),
            pltpu.VMEM((OUTW * _CHUNK,), jnp.int32),
            pltpu.VMEM((_CHUNK, D), jnp.float32),
            pltpu.VMEM((_CHUNK, D), jnp.float32),
            pltpu.VMEM((NEG * _CHUNK, D), jnp.float32),
            pltpu.VMEM((OUTW * _CHUNK + _LANES,), jnp.float32),
            pltpu.VMEM((_CHUNK * OUTW,), jnp.float32),
            pltpu.SemaphoreType.DMA,
        ],
    )


def _softplus_body(x_ref, o_ref):
    x = x_ref[...]
    o_ref[...] = jnp.maximum(x, 0.0) + jnp.log1p(jnp.exp(-jnp.abs(x)))


@functools.lru_cache(maxsize=None)
def _make_softplus(rows, cols):
    return pl.pallas_call(
        _softplus_body,
        out_shape=jax.ShapeDtypeStruct((rows, cols), jnp.float32),
    )


def kernel(inputs, labels, sampled, syn0, syn1, biases):
    B, = inputs.shape
    NEG = sampled.shape[1]
    V, D = syn0.shape
    bidx = jnp.concatenate([labels[:, None], sampled], axis=1).reshape(-1)
    logits = _make_sc_logits(B, V, D, NEG)(
        inputs, labels, sampled.reshape(B * NEG), bidx, syn0, syn1, biases)
    n = B * (NEG + 1)
    loss = _make_softplus(n // _CHUNK, _CHUNK)(
        logits.reshape(n // _CHUNK, _CHUNK))
    return loss.reshape(B, NEG + 1)


# speedup vs baseline: 1.0012x; 1.0012x over previous
import functools

import jax
import jax.numpy as jnp
from jax import lax
from jax.experimental import pallas as pl
from jax.experimental.pallas import tpu as pltpu
from jax.experimental.pallas import tpu_sc as plsc

_LANES = 16
_CHUNK = 128


@functools.lru_cache(maxsize=None)
def _make_sc_logits(B, V, D, NEG):
    info = plsc.get_sparse_core_info()
    NC, NS = info.num_cores, info.num_subcores
    NW = NC * NS
    per_w = B // NW
    n_chunks = per_w // _CHUNK
    OUTW = NEG + 1
    H = D // _LANES
    mesh = plsc.VectorSubcoreMesh(core_axis_name="c", subcore_axis_name="s")

    def body(inputs_hbm, labels_hbm, sampled_hbm,
             syn0_hbm, syn1_hbm, biases_hbm, out_hbm,
             in_idx, lab_idx, samp_idx,
             a_rows, t_rows, s_rows, lab_bias, samp_bias, out_v, sem):
        wid = lax.axis_index("s") * NC + lax.axis_index("c")
        lane = lax.iota(jnp.int32, _LANES)
        sign = jnp.where(lane == 0, -1.0, 1.0).astype(jnp.float32)

        for c in range(n_chunks):
            base = wid * per_w + c * _CHUNK
            pltpu.sync_copy(inputs_hbm.at[pl.ds(base, _CHUNK)], in_idx)
            pltpu.sync_copy(labels_hbm.at[pl.ds(base, _CHUNK)], lab_idx)
            pltpu.sync_copy(
                sampled_hbm.at[pl.ds(base * NEG, NEG * _CHUNK)], samp_idx)
            cps = [
                pltpu.async_copy(syn0_hbm.at[in_idx], a_rows, sem),
                pltpu.async_copy(syn1_hbm.at[lab_idx], t_rows, sem),
                pltpu.async_copy(biases_hbm.at[lab_idx], lab_bias, sem),
            ]
            for j in range(NEG):
                cps.append(pltpu.async_copy(
                    syn1_hbm.at[samp_idx.at[pl.ds(j * _CHUNK, _CHUNK)]],
                    s_rows.at[pl.ds(j * _CHUNK, _CHUNK)], sem))
                cps.append(pltpu.async_copy(
                    biases_hbm.at[samp_idx.at[pl.ds(j * _CHUNK, _CHUNK)]],
                    samp_bias.at[pl.ds(j * _CHUNK, _CHUNK)], sem))
            for cp in cps:
                cp.wait()

            def elem(e, carry):
                a = [a_rows[e, pl.ds(h * _LANES, _LANES)] for h in range(H)]
                p = a[0] * t_rows[e, pl.ds(0, _LANES)]
                for h in range(1, H):
                    p = p + a[h] * t_rows[e, pl.ds(h * _LANES, _LANES)]
                res = jnp.where(lane == 0, jnp.sum(p), 0.0)
                for t in range(NEG):
                    r = e * NEG + t
                    p = a[0] * s_rows[r, pl.ds(0, _LANES)]
                    for h in range(1, H):
                        p = p + a[h] * s_rows[r, pl.ds(h * _LANES, _LANES)]
                    res = jnp.where(lane == t + 1, jnp.sum(p), res)
                lab_g = plsc.load_gather(lab_bias, [lane * 0 + e])
                samp_g = plsc.load_gather(
                    samp_bias, [e * NEG + jnp.clip(lane - 1, 0, NEG - 1)])
                bvec = jnp.where(lane == 0, lab_g, samp_g)
                plsc.store_scatter(out_v, [e * OUTW + lane],
                                   sign * (res + bvec), mask=lane < OUTW)
                return carry

            lax.fori_loop(0, _CHUNK, elem, 0)
            pltpu.sync_copy(out_v,
                            out_hbm.at[pl.ds(base * OUTW, _CHUNK * OUTW)])

    return pl.kernel(
        body,
        mesh=mesh,
        out_type=jax.ShapeDtypeStruct((B * OUTW,), jnp.float32),
        compiler_params=pltpu.CompilerParams(
            needs_layout_passes=False, use_tc_tiling_on_sc=False),
        scratch_types=[
            pltpu.VMEM((_CHUNK,), jnp.int32),
            pltpu.VMEM((_CHUNK,), jnp.int32),
            pltpu.VMEM((NEG * _CHUNK,), jnp.int32),
            pltpu.VMEM((_CHUNK, D), jnp.float32),
            pltpu.VMEM((_CHUNK, D), jnp.float32),
            pltpu.VMEM((NEG * _CHUNK, D), jnp.float32),
            pltpu.VMEM((_CHUNK,), jnp.float32),
            pltpu.VMEM((NEG * _CHUNK,), jnp.float32),
            pltpu.VMEM((_CHUNK * OUTW,), jnp.float32),
            pltpu.SemaphoreType.DMA,
        ],
    )


def _softplus_body(x_ref, o_ref):
    x = x_ref[...]
    o_ref[...] = jnp.maximum(x, 0.0) + jnp.log1p(jnp.exp(-jnp.abs(x)))


@functools.lru_cache(maxsize=None)
def _make_softplus(rows, cols):
    return pl.pallas_call(
        _softplus_body,
        out_shape=jax.ShapeDtypeStruct((rows, cols), jnp.float32),
    )


def kernel(inputs, labels, sampled, syn0, syn1, biases):
    B, = inputs.shape
    NEG = sampled.shape[1]
    V, D = syn0.shape
    logits = _make_sc_logits(B, V, D, NEG)(
        inputs, labels, sampled.reshape(B * NEG), syn0, syn1, biases)
    n = B * (NEG + 1)
    loss = _make_softplus(n // _CHUNK, _CHUNK)(
        logits.reshape(n // _CHUNK, _CHUNK))
    return loss.reshape(B, NEG + 1)
